# baseline (speedup 1.0000x reference)
import functools

import jax
import jax.numpy as jnp
from jax.experimental import pallas as pl
from jax.experimental.pallas import tpu as pltpu

_N = 4096
_D = 768
_E = 8
_H = 512
_TOP_K = 2
_DIVERSITY_PENALTY = 0.01

_T = 512


def _moe_block_kernel(x_ref, gw_ref, gb_ref, w1_ref, b1_ref, w2_ref, b2_ref,
                      out_ref, usage_ref, x_scr, comb_scr, acc_scr):
    e = pl.program_id(0)
    i = pl.program_id(1)
    rows = pl.ds(i * _T, _T)

    @pl.when(e == 0)
    def _gate():
        xb = x_ref[...]
        x_scr[rows, :] = xb
        logits = jnp.dot(xb, gw_ref[...], preferred_element_type=jnp.float32)
        logits = logits + gb_ref[...]
        s = jax.nn.softmax(logits, axis=-1)

        eids = jax.lax.broadcasted_iota(jnp.int32, s.shape, 1)
        i1 = jnp.argmax(s, axis=-1)
        s1 = jnp.max(s, axis=-1)
        s_masked = jnp.where(eids == i1[:, None], -jnp.inf, s)
        i2 = jnp.argmax(s_masked, axis=-1)
        s2 = jnp.max(s_masked, axis=-1)
        denom = s1 + s2
        comb_scr[rows, :] = (
            jnp.where(eids == i1[:, None], (s1 / denom)[:, None], 0.0)
            + jnp.where(eids == i2[:, None], (s2 / denom)[:, None], 0.0))

        usum = jnp.sum(s, axis=0).reshape(1, 1, _E)

        @pl.when(i == 0)
        def _init_usage():
            usage_ref[...] = jnp.zeros_like(usage_ref)

        usage_ref[...] += usum

    x = x_scr[rows, :]
    c = comb_scr[rows, :]
    lane_e = jax.lax.broadcasted_iota(jnp.int32, c.shape, 1)
    ce = jnp.sum(jnp.where(lane_e == e, c, 0.0), axis=1)

    h = jnp.dot(x, w1_ref[0], preferred_element_type=jnp.float32)
    h = jnp.maximum(h + b1_ref[0, 0][None, :], 0.0)
    y = jnp.dot(h, w2_ref[0], preferred_element_type=jnp.float32)
    y = y + b2_ref[0, 0][None, :]
    contrib = ce[:, None] * y

    @pl.when(e == 0)
    def _init_acc():
        acc_scr[rows, :] = contrib

    @pl.when((e > 0) & (e < _E - 1))
    def _accum():
        acc_scr[rows, :] += contrib

    @pl.when(e == _E - 1)
    def _final():
        out_ref[...] = acc_scr[rows, :] + contrib


@functools.partial(jax.jit, static_argnames=())
def kernel(x, gate_W, gate_b, W1, b1, W2, b2):
    nb = _N // _T
    out, usage = pl.pallas_call(
        _moe_block_kernel,
        grid=(_E, nb),
        in_specs=[
            pl.BlockSpec((_T, _D), lambda e, i: (jnp.where(e == 0, i, 0), 0)),
            pl.BlockSpec((_D, _E), lambda e, i: (0, 0)),
            pl.BlockSpec((1, _E), lambda e, i: (0, 0)),
            pl.BlockSpec((1, _D, _H), lambda e, i: (e, 0, 0)),
            pl.BlockSpec((1, 1, _H), lambda e, i: (e, 0, 0)),
            pl.BlockSpec((1, _H, _D), lambda e, i: (e, 0, 0)),
            pl.BlockSpec((1, 1, _D), lambda e, i: (e, 0, 0)),
        ],
        out_specs=[
            pl.BlockSpec((_T, _D),
                         lambda e, i: (jnp.where(e == _E - 1, i, 0), 0)),
            pl.BlockSpec((1, 1, _E), lambda e, i: (0, 0, 0)),
        ],
        out_shape=[
            jax.ShapeDtypeStruct((_N, _D), jnp.float32),
            jax.ShapeDtypeStruct((1, 1, _E), jnp.float32),
        ],
        scratch_shapes=[
            pltpu.VMEM((_N, _D), jnp.float32),
            pltpu.VMEM((_N, _E), jnp.float32),
            pltpu.VMEM((_N, _D), jnp.float32),
        ],
    )(x, gate_W, gate_b.reshape(1, _E), W1, b1.reshape(_E, 1, _H), W2,
      b2.reshape(_E, 1, _D))
    expert_usage = usage[0, 0] / _N
    load_balance_loss = _DIVERSITY_PENALTY * jnp.sum(expert_usage ** 2)
    return (out, load_balance_loss)


# speedup vs baseline: 1.2694x; 1.2694x over previous
import functools

import jax
import jax.numpy as jnp
from jax.experimental import pallas as pl
from jax.experimental.pallas import tpu as pltpu

_N = 4096
_D = 768
_E = 8
_H = 512
_TOP_K = 2
_DIVERSITY_PENALTY = 0.01

_T = 512
_G = 2
_EPG = _E // _G


def _moe_block_kernel(x_ref, gw_ref, gb_ref, w1_ref, b1_ref, w2_ref, b2_ref,
                      out_ref, usage_ref, comb_scr, acc_scr):
    g = pl.program_id(0)
    i = pl.program_id(1)
    rows = pl.ds(i * _T, _T)
    xb = x_ref[...]

    @pl.when(g == 0)
    def _gate():
        logits = jnp.dot(xb, gw_ref[...], preferred_element_type=jnp.float32)
        logits = logits + gb_ref[...]
        s = jax.nn.softmax(logits, axis=-1)

        eids = jax.lax.broadcasted_iota(jnp.int32, s.shape, 1)
        i1 = jnp.argmax(s, axis=-1)
        s1 = jnp.max(s, axis=-1)
        s_masked = jnp.where(eids == i1[:, None], -jnp.inf, s)
        i2 = jnp.argmax(s_masked, axis=-1)
        s2 = jnp.max(s_masked, axis=-1)
        denom = s1 + s2
        comb_scr[rows, :] = (
            jnp.where(eids == i1[:, None], (s1 / denom)[:, None], 0.0)
            + jnp.where(eids == i2[:, None], (s2 / denom)[:, None], 0.0))

        @pl.when(i == 0)
        def _init_usage():
            usage_ref[...] = jnp.zeros_like(usage_ref)

        usage_ref[...] += jnp.sum(s, axis=0).reshape(1, 1, _E)

    c = comb_scr[rows, :]
    lane_e = jax.lax.broadcasted_iota(jnp.int32, c.shape, 1)

    partial = jnp.zeros((_T, _D), jnp.float32)
    for k in range(_EPG):
        ce = jnp.sum(jnp.where(lane_e == g * _EPG + k, c, 0.0), axis=1)
        h = jnp.dot(xb, w1_ref[k], preferred_element_type=jnp.float32)
        h = jnp.maximum(h + b1_ref[k, 0][None, :], 0.0)
        y = jnp.dot(h, w2_ref[k], preferred_element_type=jnp.float32)
        y = y + b2_ref[k, 0][None, :]
        partial = partial + ce[:, None] * y

    @pl.when(g == 0)
    def _init_acc():
        acc_scr[rows, :] = partial
        out_ref[...] = partial

    @pl.when(g == _G - 1)
    def _final():
        out_ref[...] = acc_scr[rows, :] + partial


@functools.partial(jax.jit, static_argnames=())
def kernel(x, gate_W, gate_b, W1, b1, W2, b2):
    nb = _N // _T
    out, usage = pl.pallas_call(
        _moe_block_kernel,
        grid=(_G, nb),
        in_specs=[
            pl.BlockSpec((_T, _D), lambda g, i: (i, 0)),
            pl.BlockSpec((_D, _E), lambda g, i: (0, 0)),
            pl.BlockSpec((1, _E), lambda g, i: (0, 0)),
            pl.BlockSpec((_EPG, _D, _H), lambda g, i: (g, 0, 0)),
            pl.BlockSpec((_EPG, 1, _H), lambda g, i: (g, 0, 0)),
            pl.BlockSpec((_EPG, _H, _D), lambda g, i: (g, 0, 0)),
            pl.BlockSpec((_EPG, 1, _D), lambda g, i: (g, 0, 0)),
        ],
        out_specs=[
            pl.BlockSpec((_T, _D), lambda g, i: (i, 0)),
            pl.BlockSpec((1, 1, _E), lambda g, i: (0, 0, 0)),
        ],
        out_shape=[
            jax.ShapeDtypeStruct((_N, _D), jnp.float32),
            jax.ShapeDtypeStruct((1, 1, _E), jnp.float32),
        ],
        scratch_shapes=[
            pltpu.VMEM((_N, _E), jnp.float32),
            pltpu.VMEM((_N, _D), jnp.float32),
        ],
    )(x, gate_W, gate_b.reshape(1, _E), W1, b1.reshape(_E, 1, _H), W2,
      b2.reshape(_E, 1, _D))
    expert_usage = usage[0, 0] / _N
    load_balance_loss = _DIVERSITY_PENALTY * jnp.sum(expert_usage ** 2)
    return (out, load_balance_loss)


# speedup vs baseline: 1.4308x; 1.1272x over previous
import functools

import jax
import jax.numpy as jnp
from jax.experimental import pallas as pl

_N = 4096
_D = 768
_E = 8
_H = 512
_TOP_K = 2
_DIVERSITY_PENALTY = 0.01

_T = 1024


def _moe_block_kernel(x_ref, gw_ref, gb_ref, w1_ref, b1_ref, w2_ref, b2_ref,
                      out_ref, usage_ref):
    x = x_ref[...]
    logits = jnp.dot(x, gw_ref[...], preferred_element_type=jnp.float32)
    logits = logits + gb_ref[...]
    s = jax.nn.softmax(logits, axis=-1)
    usage_ref[0, :, :] = jnp.sum(s, axis=0, keepdims=True)

    eids = jax.lax.broadcasted_iota(jnp.int32, s.shape, 1)
    i1 = jnp.argmax(s, axis=-1)
    s1 = jnp.max(s, axis=-1)
    s_masked = jnp.where(eids == i1[:, None], -jnp.inf, s)
    i2 = jnp.argmax(s_masked, axis=-1)
    s2 = jnp.max(s_masked, axis=-1)
    denom = s1 + s2
    combine = (jnp.where(eids == i1[:, None], (s1 / denom)[:, None], 0.0)
               + jnp.where(eids == i2[:, None], (s2 / denom)[:, None], 0.0))

    acc = jnp.zeros((x.shape[0], _D), jnp.float32)
    for e in range(_E):
        h = jnp.dot(x, w1_ref[e], preferred_element_type=jnp.float32)
        h = jnp.maximum(h + b1_ref[e][None, :], 0.0)
        y = jnp.dot(h, w2_ref[e], preferred_element_type=jnp.float32)
        y = y + b2_ref[e][None, :]
        acc = acc + combine[:, e][:, None] * y
    out_ref[...] = acc


@functools.partial(jax.jit, static_argnames=())
def kernel(x, gate_W, gate_b, W1, b1, W2, b2):
    nb = _N // _T
    out, usage = pl.pallas_call(
        _moe_block_kernel,
        grid=(nb,),
        in_specs=[
            pl.BlockSpec((_T, _D), lambda i: (i, 0)),
            pl.BlockSpec((_D, _E), lambda i: (0, 0)),
            pl.BlockSpec((1, _E), lambda i: (0, 0)),
            pl.BlockSpec((_E, _D, _H), lambda i: (0, 0, 0)),
            pl.BlockSpec((_E, _H), lambda i: (0, 0)),
            pl.BlockSpec((_E, _H, _D), lambda i: (0, 0, 0)),
            pl.BlockSpec((_E, _D), lambda i: (0, 0)),
        ],
        out_specs=[
            pl.BlockSpec((_T, _D), lambda i: (i, 0)),
            pl.BlockSpec((1, 1, _E), lambda i: (i, 0, 0)),
        ],
        out_shape=[
            jax.ShapeDtypeStruct((_N, _D), jnp.float32),
            jax.ShapeDtypeStruct((nb, 1, _E), jnp.float32),
        ],
    )(x, gate_W, gate_b.reshape(1, _E), W1, b1, W2, b2)
    expert_usage = jnp.sum(usage, axis=(0, 1)) / _N
    load_balance_loss = _DIVERSITY_PENALTY * jnp.sum(expert_usage ** 2)
    return (out, load_balance_loss)
